# baseline (speedup 1.0000x reference)
import functools

import jax
import jax.numpy as jnp
from jax import lax
from jax.experimental import pallas as pl
from jax.experimental.pallas import tpu as pltpu
from jax.experimental.pallas import tpu_sc as plsc

NUM_NODES = 1000000
EMBED_DIM = 64
BATCH = 16384

_info = plsc.get_sparse_core_info()
_NC = _info.num_cores
_NS = _info.num_subcores
_NW = _NC * _NS
_B_PER_W = BATCH // _NW


def _gather_kernel(table_hbm, idx_hbm, out_hbm, idx_v, rows_v, sem):
    wid = lax.axis_index("s") * _NC + lax.axis_index("c")
    base = wid * _B_PER_W
    pltpu.sync_copy(idx_hbm.at[pl.ds(base, _B_PER_W)], idx_v)
    pltpu.async_copy(table_hbm.at[idx_v], rows_v, sem).wait()
    pltpu.sync_copy(rows_v, out_hbm.at[pl.ds(base, _B_PER_W)])


@jax.jit
def _gather(table, idx):
    mesh = plsc.VectorSubcoreMesh(core_axis_name="c", subcore_axis_name="s")
    return pl.kernel(
        _gather_kernel,
        mesh=mesh,
        out_type=jax.ShapeDtypeStruct((BATCH, EMBED_DIM), jnp.float32),
        scratch_types=[
            pltpu.VMEM((_B_PER_W,), jnp.int32),
            pltpu.VMEM((_B_PER_W, EMBED_DIM), jnp.float32),
            pltpu.SemaphoreType.DMA,
        ],
        compiler_params=pltpu.CompilerParams(use_tc_tiling_on_sc=False),
    )(table, idx)


def kernel(batch, embedding_weight):
    return _gather(embedding_weight, batch.astype(jnp.int32))


# speedup vs baseline: 1.5398x; 1.5398x over previous
import jax
import jax.numpy as jnp
from jax import lax
from jax.experimental import pallas as pl
from jax.experimental.pallas import tpu as pltpu
from jax.experimental.pallas import tpu_sc as plsc

NUM_NODES = 1000000
EMBED_DIM = 64
BATCH = 16384

_info = plsc.get_sparse_core_info()
_NC = _info.num_cores
_NS = _info.num_subcores
_NW = _NC * _NS
_B_PER_W = BATCH // _NW
_K = 8


def _gather_kernel(table_hbm, idx_hbm, out_hbm, idx_v, bufs_v, rows_v, sems, wsem):
    wid = lax.axis_index("s") * _NC + lax.axis_index("c")
    base = wid * _B_PER_W
    pltpu.sync_copy(idx_hbm.at[pl.ds(base, _B_PER_W)], idx_v.at[pl.ds(0, _B_PER_W)])
    lanes = lax.iota(jnp.int32, 16)
    zeros = jnp.zeros((16,), jnp.int32)

    def fire(n_scalar, k):
        n8 = pl.multiple_of((n_scalar // 8) * 8, 8)
        pltpu.async_copy(
            table_hbm.at[pl.ds(n8, 8), :], bufs_v.at[k], sems.at[k]
        )

    def extract(n_scalar, g, k):
        r = zeros + (n_scalar - (n_scalar // 8) * 8)
        gv = zeros + g
        for grp in range(4):
            vals = plsc.load_gather(bufs_v.at[k], [r, grp * 16 + lanes])
            plsc.store_scatter(rows_v, [gv, grp * 16 + lanes], vals)

    v0 = idx_v[pl.ds(0, 16)]
    for k in range(_K):
        fire(v0[k], k)

    def body(o, carry):
        vcur = idx_v[pl.ds(o * 16, 16)]
        vnxt = idx_v[pl.ds(o * 16 + _K, 16)]
        for j in range(16):
            g = o * 16 + j
            k = j % _K
            pltpu.make_async_copy(
                table_hbm.at[pl.ds(0, 8), :], bufs_v.at[k], sems.at[k]
            ).wait()
            extract(vcur[j], g, k)

            @pl.when(g + _K < _B_PER_W)
            def _(j=j, k=k, vnxt=vnxt):
                fire(vnxt[j], k)

        return carry

    lax.fori_loop(0, _B_PER_W // 16, body, 0)
    pltpu.async_copy(rows_v, out_hbm.at[pl.ds(base, _B_PER_W)], wsem).wait()


@jax.jit
def _gather(table, idx):
    mesh = plsc.VectorSubcoreMesh(core_axis_name="c", subcore_axis_name="s")
    return pl.kernel(
        _gather_kernel,
        mesh=mesh,
        out_type=jax.ShapeDtypeStruct((BATCH, EMBED_DIM), jnp.float32),
        scratch_types=[
            pltpu.VMEM((_B_PER_W + 16,), jnp.int32),
            pltpu.VMEM((_K, 8, EMBED_DIM), jnp.float32),
            pltpu.VMEM((_B_PER_W, EMBED_DIM), jnp.float32),
            pltpu.SemaphoreType.DMA((_K,)),
            pltpu.SemaphoreType.DMA,
        ],
        compiler_params=pltpu.CompilerParams(
            use_tc_tiling_on_sc=True, needs_layout_passes=False
        ),
    )(table, idx)


def kernel(batch, embedding_weight):
    return _gather(embedding_weight, batch.astype(jnp.int32))


# speedup vs baseline: 1.5670x; 1.0176x over previous
import jax
import jax.numpy as jnp
from jax import lax
from jax.experimental import pallas as pl
from jax.experimental.pallas import tpu as pltpu
from jax.experimental.pallas import tpu_sc as plsc

NUM_NODES = 1000000
EMBED_DIM = 64
BATCH = 16384

_info = plsc.get_sparse_core_info()
_NC = _info.num_cores
_NS = _info.num_subcores
_NW = _NC * _NS
_B_PER_W = BATCH // _NW
_K = 8


def _gather_kernel(table_hbm, idx_hbm, outt_hbm, idx_v, bufs_v, cols_v, sems, wsem):
    wid = lax.axis_index("s") * _NC + lax.axis_index("c")
    base = wid * _B_PER_W
    pltpu.sync_copy(idx_hbm.at[pl.ds(base, _B_PER_W)], idx_v.at[pl.ds(0, _B_PER_W)])
    lanes = lax.iota(jnp.int32, 16)
    zeros = jnp.zeros((16,), jnp.int32)

    def fire(n_scalar, k):
        n8 = pl.multiple_of((n_scalar // 8) * 8, 8)
        pltpu.async_copy(
            table_hbm.at[pl.ds(n8, 8), :], bufs_v.at[k], sems.at[k]
        )

    def extract(n_scalar, g, k):
        r = zeros + (n_scalar - (n_scalar // 8) * 8)
        gv = zeros + g
        for grp in range(4):
            vals = plsc.load_gather(bufs_v.at[k], [r, grp * 16 + lanes])
            plsc.store_scatter(cols_v, [grp * 16 + lanes, gv], vals)

    v0 = idx_v[pl.ds(0, 16)]
    for k in range(_K):
        fire(v0[k], k)

    def body(o, carry):
        vcur = idx_v[pl.ds(o * 16, 16)]
        vnxt = idx_v[pl.ds(o * 16 + _K, 16)]
        for j in range(16):
            g = o * 16 + j
            k = j % _K
            pltpu.make_async_copy(
                table_hbm.at[pl.ds(0, 8), :], bufs_v.at[k], sems.at[k]
            ).wait()
            extract(vcur[j], g, k)

            @pl.when(g + _K < _B_PER_W)
            def _(j=j, k=k, vnxt=vnxt):
                fire(vnxt[j], k)

        return carry

    lax.fori_loop(0, _B_PER_W // 16, body, 0)
    pltpu.async_copy(cols_v, outt_hbm.at[:, pl.ds(base, _B_PER_W)], wsem).wait()


@jax.jit
def _gather(table, idx):
    mesh = plsc.VectorSubcoreMesh(core_axis_name="c", subcore_axis_name="s")
    return pl.kernel(
        _gather_kernel,
        mesh=mesh,
        out_type=jax.ShapeDtypeStruct((EMBED_DIM, BATCH), jnp.float32),
        scratch_types=[
            pltpu.VMEM((_B_PER_W + 16,), jnp.int32),
            pltpu.VMEM((_K, 8, EMBED_DIM), jnp.float32),
            pltpu.VMEM((EMBED_DIM, _B_PER_W), jnp.float32),
            pltpu.SemaphoreType.DMA((_K,)),
            pltpu.SemaphoreType.DMA,
        ],
        compiler_params=pltpu.CompilerParams(
            use_tc_tiling_on_sc=True, needs_layout_passes=False
        ),
    )(table, idx)


def kernel(batch, embedding_weight):
    return _gather(embedding_weight, batch.astype(jnp.int32)).T


# speedup vs baseline: 3.0127x; 1.9226x over previous
import jax
import jax.numpy as jnp
from jax import lax
from jax.experimental import pallas as pl
from jax.experimental.pallas import tpu as pltpu
from jax.experimental.pallas import tpu_sc as plsc

NUM_NODES = 1000000
EMBED_DIM = 64
BATCH = 16384

_info = plsc.get_sparse_core_info()
_NC = _info.num_cores
_NS = _info.num_subcores
_NW = _NC * _NS
_B_PER_W = BATCH // _NW
_K = 8


def _gather_kernel(tablet_hbm, idx_hbm, outt_hbm, idx_v, bufs_v, cols_v, sems, wsem):
    wid = lax.axis_index("s") * _NC + lax.axis_index("c")
    base = wid * _B_PER_W
    pltpu.sync_copy(idx_hbm.at[pl.ds(base, _B_PER_W)], idx_v.at[pl.ds(0, _B_PER_W)])
    lanes = lax.iota(jnp.int32, 16)
    zeros = jnp.zeros((16,), jnp.int32)

    def slab_base(n_scalar):
        c128 = (n_scalar // 128) * 128
        return pl.multiple_of(lax.min(c128, NUM_NODES - 128), 128)

    def fire(n_scalar, k):
        pltpu.async_copy(
            tablet_hbm.at[:, pl.ds(slab_base(n_scalar), 128)],
            bufs_v.at[k],
            sems.at[k],
        )

    def extract(n_scalar, g, k):
        cv = zeros + (n_scalar - slab_base(n_scalar))
        gv = zeros + g
        for grp in range(4):
            vals = plsc.load_gather(bufs_v.at[k], [grp * 16 + lanes, cv])
            plsc.store_scatter(cols_v, [grp * 16 + lanes, gv], vals)

    v0 = idx_v[pl.ds(0, 16)]
    for k in range(_K):
        fire(v0[k], k)

    def body(o, carry):
        vcur = idx_v[pl.ds(o * 16, 16)]
        vnxt = idx_v[pl.ds(o * 16 + _K, 16)]
        for j in range(16):
            g = o * 16 + j
            k = j % _K
            pltpu.make_async_copy(
                tablet_hbm.at[:, pl.ds(0, 128)], bufs_v.at[k], sems.at[k]
            ).wait()
            extract(vcur[j], g, k)

            @pl.when(g + _K < _B_PER_W)
            def _(j=j, k=k, vnxt=vnxt):
                fire(vnxt[j], k)

        return carry

    lax.fori_loop(0, _B_PER_W // 16, body, 0)
    pltpu.async_copy(cols_v, outt_hbm.at[:, pl.ds(base, _B_PER_W)], wsem).wait()


@jax.jit
def _gather(table, idx):
    mesh = plsc.VectorSubcoreMesh(core_axis_name="c", subcore_axis_name="s")
    return pl.kernel(
        _gather_kernel,
        mesh=mesh,
        out_type=jax.ShapeDtypeStruct((EMBED_DIM, BATCH), jnp.float32),
        scratch_types=[
            pltpu.VMEM((_B_PER_W + 16,), jnp.int32),
            pltpu.VMEM((_K, EMBED_DIM, 128), jnp.float32),
            pltpu.VMEM((EMBED_DIM, _B_PER_W), jnp.float32),
            pltpu.SemaphoreType.DMA((_K,)),
            pltpu.SemaphoreType.DMA,
        ],
        compiler_params=pltpu.CompilerParams(
            use_tc_tiling_on_sc=True, needs_layout_passes=False
        ),
    )(table, idx)


def kernel(batch, embedding_weight):
    return _gather(embedding_weight.T, batch.astype(jnp.int32)).T


# speedup vs baseline: 3.0204x; 1.0026x over previous
import jax
import jax.numpy as jnp
from jax import lax
from jax.experimental import pallas as pl
from jax.experimental.pallas import tpu as pltpu
from jax.experimental.pallas import tpu_sc as plsc

NUM_NODES = 1000000
EMBED_DIM = 64
BATCH = 16384

_info = plsc.get_sparse_core_info()
_NC = _info.num_cores
_NS = _info.num_subcores
_NW = _NC * _NS
_B_PER_W = BATCH // _NW
_K = 8


def _gather_kernel(tablet_hbm, idx_hbm, outt_hbm, idx_v, bufs_v, cols_v, sems, wsem):
    wid = lax.axis_index("s") * _NC + lax.axis_index("c")
    base = wid * _B_PER_W
    pltpu.sync_copy(idx_hbm.at[pl.ds(base, _B_PER_W)], idx_v.at[pl.ds(0, _B_PER_W)])
    lanes = lax.iota(jnp.int32, 16)
    zeros = jnp.zeros((16,), jnp.int32)

    def slab_base(n_scalar):
        return pl.multiple_of((n_scalar // 128) * 128, 128)

    def fire(n_scalar, k):
        pltpu.async_copy(
            tablet_hbm.at[:, pl.ds(slab_base(n_scalar), 128)],
            bufs_v.at[k],
            sems.at[k],
        )

    def extract(n_scalar, g, k):
        cv = zeros + (n_scalar - slab_base(n_scalar))
        gv = zeros + g
        for grp in range(4):
            vals = plsc.load_gather(bufs_v.at[k], [grp * 16 + lanes, cv])
            plsc.store_scatter(cols_v, [grp * 16 + lanes, gv], vals)

    v0 = idx_v[pl.ds(0, 16)]
    for k in range(_K):
        fire(v0[k], k)

    def body(o, carry):
        vcur = idx_v[pl.ds(o * 16, 16)]
        vnxt = idx_v[pl.ds(o * 16 + _K, 16)]
        for j in range(16):
            g = o * 16 + j
            k = j % _K
            pltpu.make_async_copy(
                tablet_hbm.at[:, pl.ds(0, 128)], bufs_v.at[k], sems.at[k]
            ).wait()
            extract(vcur[j], g, k)

            @pl.when(g + _K < _B_PER_W)
            def _(j=j, k=k, vnxt=vnxt):
                fire(vnxt[j], k)

        return carry

    lax.fori_loop(0, _B_PER_W // 16, body, 0)
    pltpu.async_copy(cols_v, outt_hbm.at[:, pl.ds(base, _B_PER_W)], wsem).wait()


@jax.jit
def _gather(table, idx):
    mesh = plsc.VectorSubcoreMesh(core_axis_name="c", subcore_axis_name="s")
    return pl.kernel(
        _gather_kernel,
        mesh=mesh,
        out_type=jax.ShapeDtypeStruct((EMBED_DIM, BATCH), jnp.float32),
        scratch_types=[
            pltpu.VMEM((_B_PER_W + 16,), jnp.int32),
            pltpu.VMEM((_K, EMBED_DIM, 128), jnp.float32),
            pltpu.VMEM((EMBED_DIM, _B_PER_W), jnp.float32),
            pltpu.SemaphoreType.DMA((_K,)),
            pltpu.SemaphoreType.DMA,
        ],
        compiler_params=pltpu.CompilerParams(
            use_tc_tiling_on_sc=True, needs_layout_passes=False
        ),
    )(table, idx)


def kernel(batch, embedding_weight):
    return _gather(embedding_weight.T, batch.astype(jnp.int32)).T
